# Initial kernel scaffold; baseline (speedup 1.0000x reference)
import functools

import jax
import jax.numpy as jnp
from jax import lax
from jax.experimental import pallas as pl
from jax.experimental.pallas import tpu as pltpu
from jax.experimental.pallas import tpu_sc as plsc

VOCAB = 256
HIDDEN = 5
CTX = 8
HPAD = 16
FPAD = CTX * HPAD
NW = 32


@functools.lru_cache(maxsize=None)
def _make_sc_gather(n_lookups: int):
    per_w = n_lookups // NW
    mesh = plsc.VectorSubcoreMesh(core_axis_name="c", subcore_axis_name="s")

    @functools.partial(
        pl.kernel,
        out_type=jax.ShapeDtypeStruct((n_lookups, HPAD), jnp.float32),
        mesh=mesh,
        scratch_types=[
            pltpu.VMEM((per_w,), jnp.int32),
            pltpu.VMEM((per_w, HPAD), jnp.float32),
            pltpu.SemaphoreType.DMA,
        ],
    )
    def sc_gather(idx_hbm, table_hbm, out_hbm, idx_v, rows_v, sem):
        wid = lax.axis_index("s") * 2 + lax.axis_index("c")
        base = wid * per_w
        pltpu.sync_copy(idx_hbm.at[pl.ds(base, per_w)], idx_v)
        pltpu.async_copy(table_hbm.at[idx_v], rows_v, sem).wait()
        pltpu.sync_copy(rows_v, out_hbm.at[pl.ds(base, per_w)])

    return sc_gather


def _dense_body(emb_ref, w_ref, b_ref, out_ref):
    out_ref[...] = lax.dot_general(
        emb_ref[...], w_ref[...], (((1,), (1,)), ((), ())),
        preferred_element_type=jnp.float32) + b_ref[...]


def _dense(emb, w_pad, b2d, batch: int, tile: int):
    grid = (batch // tile,)
    return pl.pallas_call(
        _dense_body,
        grid=grid,
        in_specs=[
            pl.BlockSpec((tile, FPAD), lambda i: (i, 0)),
            pl.BlockSpec((VOCAB, FPAD), lambda i: (0, 0)),
            pl.BlockSpec((1, VOCAB), lambda i: (0, 0)),
        ],
        out_specs=pl.BlockSpec((tile, VOCAB), lambda i: (i, 0)),
        out_shape=jax.ShapeDtypeStruct((batch, VOCAB), jnp.float32),
    )(emb, w_pad, b2d)


def kernel(x, embed_table, fc_w, fc_b):
    batch, ctx = x.shape
    vocab, hidden = embed_table.shape

    table_pad = jnp.pad(embed_table, ((0, 0), (0, HPAD - hidden)))
    w_pad = jnp.pad(
        fc_w.reshape(vocab, ctx, hidden), ((0, 0), (0, 0), (0, HPAD - hidden))
    ).reshape(vocab, ctx * HPAD)

    emb = _make_sc_gather(batch * ctx)(x.reshape(-1), table_pad)
    emb2 = emb.reshape(batch, ctx * HPAD)
    return _dense(emb2, w_pad, fc_b.reshape(1, vocab), batch, tile=2048)


# speedup vs baseline: 7.2344x; 7.2344x over previous
import functools

import jax
import jax.numpy as jnp
from jax import lax
from jax.experimental import pallas as pl
from jax.experimental.pallas import tpu as pltpu
from jax.experimental.pallas import tpu_sc as plsc

VOCAB = 256
HIDDEN = 5
CTX = 8
HPAD = 16
FPAD = CTX * HPAD
NW = 32


@functools.lru_cache(maxsize=None)
def _make_sc_gather(n_lookups: int):
    per_w = n_lookups // NW
    mesh = plsc.VectorSubcoreMesh(core_axis_name="c", subcore_axis_name="s")

    @functools.partial(
        pl.kernel,
        out_type=jax.ShapeDtypeStruct((n_lookups, HPAD), jnp.float32),
        mesh=mesh,
        scratch_types=[
            pltpu.VMEM((per_w,), jnp.int32),
            pltpu.VMEM((per_w, HPAD), jnp.float32),
            pltpu.SemaphoreType.DMA,
        ],
        compiler_params=pltpu.CompilerParams(use_tc_tiling_on_sc=False),
    )
    def sc_gather(idx_hbm, table_hbm, out_hbm, idx_v, rows_v, sem):
        wid = lax.axis_index("s") * 2 + lax.axis_index("c")
        base = wid * per_w
        pltpu.sync_copy(idx_hbm.at[pl.ds(base, per_w)], idx_v)
        pltpu.async_copy(table_hbm.at[idx_v], rows_v, sem).wait()
        pltpu.sync_copy(rows_v, out_hbm.at[pl.ds(base, per_w)])

    return sc_gather


def _dense_body(emb_ref, w_ref, b_ref, out_ref):
    out_ref[...] = lax.dot_general(
        emb_ref[...], w_ref[...], (((1,), (1,)), ((), ())),
        preferred_element_type=jnp.float32) + b_ref[...]


def _dense(emb, w_pad, b2d, batch: int, tile: int):
    grid = (batch // tile,)
    return pl.pallas_call(
        _dense_body,
        grid=grid,
        in_specs=[
            pl.BlockSpec((tile, FPAD), lambda i: (i, 0)),
            pl.BlockSpec((VOCAB, FPAD), lambda i: (0, 0)),
            pl.BlockSpec((1, VOCAB), lambda i: (0, 0)),
        ],
        out_specs=pl.BlockSpec((tile, VOCAB), lambda i: (i, 0)),
        out_shape=jax.ShapeDtypeStruct((batch, VOCAB), jnp.float32),
    )(emb, w_pad, b2d)


def kernel(x, embed_table, fc_w, fc_b):
    batch, ctx = x.shape
    vocab, hidden = embed_table.shape

    table_pad = jnp.pad(embed_table, ((0, 0), (0, HPAD - hidden)))
    w_pad = jnp.pad(
        fc_w.reshape(vocab, ctx, hidden), ((0, 0), (0, 0), (0, HPAD - hidden))
    ).reshape(vocab, ctx * HPAD)

    emb = _make_sc_gather(batch * ctx)(x.reshape(-1), table_pad)
    emb2 = emb.reshape(batch, ctx * HPAD)
    return _dense(emb2, w_pad, fc_b.reshape(1, vocab), batch, tile=2048)
